# baseline (speedup 1.0000x reference)
import jax
import jax.numpy as jnp
from jax.experimental import pallas as pl

H = 200
G = 256


def _leaky(x):
    return jnp.where(x >= 0, x, 0.01 * x)


def _elu(x):
    return jnp.where(x > 0, x, jnp.exp(jnp.minimum(x, 0.0)) - 1.0)


def _dot(a, b):
    return jnp.dot(a, b, preferred_element_type=jnp.float32)


def _gru_core(x, hh, w):
    wir, wiz, win, whr, whz, whn, bir, biz, bin_, bhr, bhz, bhn = w
    r = jax.nn.sigmoid(_dot(x, wir[...]) + bir[...] + _dot(hh, whr[...]) + bhr[...])
    z = jax.nn.sigmoid(_dot(x, wiz[...]) + biz[...] + _dot(hh, whz[...]) + bhz[...])
    n = jnp.tanh(_dot(x, win[...]) + bin_[...] + r * (_dot(hh, whn[...]) + bhn[...]))
    return jnp.maximum((1.0 - z) * n + z * hh, 0.0)


def _split_gru(Wih, Whh, bih, bhh):
    return (
        Wih[:H].T, Wih[H : 2 * H].T, Wih[2 * H :].T,
        Whh[:H].T, Whh[H : 2 * H].T, Whh[2 * H :].T,
        bih[:H].reshape(1, H), bih[H : 2 * H].reshape(1, H), bih[2 * H :].reshape(1, H),
        bhh[:H].reshape(1, H), bhh[H : 2 * H].reshape(1, H), bhh[2 * H :].reshape(1, H),
    )


def _full(r, c):
    return pl.BlockSpec((r, c), lambda i: (0, 0))


def _row(block, c):
    return pl.BlockSpec((block, c), lambda i: (i, 0))


def _seg_softmax(logits, seg, num_segments):
    m = jax.ops.segment_max(logits, seg, num_segments=num_segments)
    m = jnp.where(jnp.isfinite(m), m, 0.0)
    ex = jnp.exp(logits - m[seg])
    s = jax.ops.segment_sum(ex, seg, num_segments=num_segments)
    return ex / (s[seg] + 1e-12)


def _nodeA(nf, Wpn, bpn, W1a, b1, W2a, block):
    N, DN = nf.shape

    def kern(nfr, wpn, bpnr, w1a, b1r, w2a, hv_o, c1_o, ld_o):
        x = nfr[...]
        hv = _leaky(_dot(x, wpn[...]) + bpnr[...])
        hv_o[...] = hv
        c1_o[...] = _dot(x, w1a[...]) + b1r[...]
        ld_o[...] = _dot(hv, w2a[...])

    return pl.pallas_call(
        kern,
        grid=(N // block,),
        in_specs=[_row(block, DN), _full(DN, H), _full(1, H), _full(DN, H),
                  _full(1, H), _full(H, 1)],
        out_specs=[_row(block, H), _row(block, H), _row(block, 1)],
        out_shape=[
            jax.ShapeDtypeStruct((N, H), jnp.float32),
            jax.ShapeDtypeStruct((N, H), jnp.float32),
            jax.ShapeDtypeStruct((N, 1), jnp.float32),
        ],
    )(nf, Wpn, bpn, W1a, b1, W2a)


def _edgeB(c1s, ef, ldd, W1b, W2b, b2, block):
    E, DE = ef.shape

    def kern(c1r, efr, ldr, w1b, w2b, b2r, lo, he_o):
        he1 = _leaky(c1r[...] + _dot(efr[...], w1b[...]))
        lo[...] = _leaky(ldr[...] + _dot(he1, w2b[...]) + b2r[...])
        he_o[...] = he1

    return pl.pallas_call(
        kern,
        grid=(E // block,),
        in_specs=[_row(block, H), _row(block, DE), _row(block, 1),
                  _full(DE, H), _full(H, 1), _full(1, 1)],
        out_specs=[_row(block, 1), _row(block, H)],
        out_shape=[
            jax.ShapeDtypeStruct((E, 1), jnp.float32),
            jax.ShapeDtypeStruct((E, H), jnp.float32),
        ],
    )(c1s, ef, ldd, W1b, W2b, b2)


def _gruC(q, sa, hv, Wet, bet, gru_w, Wla, Wlb, Wpn2, bpn2, block):
    N = q.shape[0]

    def kern(qr, sar, hvr, wet, betr, *rest):
        gw = rest[:12]
        wla, wlb, wpn2, bpn2r = rest[12:16]
        h_o, u_o, v_o, hp_o = rest[16:]
        x = _elu(_dot(qr[...], wet[...]) + sar[...] * betr[...])
        h = _gru_core(x, hvr[...], gw)
        h_o[...] = h
        u_o[...] = _dot(h, wla[...])
        v_o[...] = _dot(h, wlb[...])
        hp_o[...] = _dot(h, wpn2[...]) + bpn2r[...]

    return pl.pallas_call(
        kern,
        grid=(N // block,),
        in_specs=[_row(block, H), _row(block, 1), _row(block, H),
                  _full(H, H), _full(1, H)]
        + [_full(H, H)] * 6 + [_full(1, H)] * 6
        + [_full(H, 1), _full(H, 1), _full(H, H), _full(1, H)],
        out_specs=[_row(block, H), _row(block, 1), _row(block, 1), _row(block, H)],
        out_shape=[
            jax.ShapeDtypeStruct((N, H), jnp.float32),
            jax.ShapeDtypeStruct((N, 1), jnp.float32),
            jax.ShapeDtypeStruct((N, 1), jnp.float32),
            jax.ShapeDtypeStruct((N, H), jnp.float32),
        ],
    )(q, sa, hv, Wet, bet, *gru_w, Wla, Wlb, Wpn2, bpn2)


def _gruD(ctx2, h, gru_w, Wz0b, Wz1b, block):
    N = ctx2.shape[0]

    def kern(cr, hr, *rest):
        gw = rest[:12]
        wz0, wz1 = rest[12:14]
        h_o, z0_o, z1_o = rest[14:]
        h2 = _gru_core(_elu(cr[...]), hr[...], gw)
        h_o[...] = h2
        z0_o[...] = _dot(h2, wz0[...])
        z1_o[...] = _dot(h2, wz1[...])

    return pl.pallas_call(
        kern,
        grid=(N // block,),
        in_specs=[_row(block, H), _row(block, H)]
        + [_full(H, H)] * 6 + [_full(1, H)] * 6 + [_full(H, 1)] * 2,
        out_specs=[_row(block, H), _row(block, 1), _row(block, 1)],
        out_shape=[
            jax.ShapeDtypeStruct((N, H), jnp.float32),
            jax.ShapeDtypeStruct((N, 1), jnp.float32),
            jax.ShapeDtypeStruct((N, 1), jnp.float32),
        ],
    )(ctx2, h, *gru_w, Wz0b, Wz1b)


def _gz(g_feats, Wza, bz):
    def kern(gr, wza, bzr, o):
        o[...] = _dot(jnp.maximum(gr[...], 0.0), wza[...]) + bzr[...]

    return pl.pallas_call(
        kern,
        grid=(1,),
        in_specs=[_row(G, H), _full(H, 1), _full(1, 1)],
        out_specs=_row(G, 1),
        out_shape=jax.ShapeDtypeStruct((G, 1), jnp.float32),
    )(g_feats, Wza, bz)


def _gruG(s3h, sa3, g_feats, Wp, bp, gru_w):
    def kern(sr, sar, gr, wp, bpr, *rest):
        gw = rest[:12]
        (o,) = rest[12:]
        ctx = _elu(_dot(sr[...], wp[...]) + sar[...] * bpr[...])
        o[...] = _gru_core(ctx, gr[...], gw)

    return pl.pallas_call(
        kern,
        grid=(1,),
        in_specs=[_row(G, H), _row(G, 1), _row(G, H), _full(H, H), _full(1, H)]
        + [_full(H, H)] * 6 + [_full(1, H)] * 6,
        out_specs=_row(G, H),
        out_shape=jax.ShapeDtypeStruct((G, H), jnp.float32),
    )(s3h, sa3, g_feats, Wp, bp, *gru_w)


def kernel(node_feats, edge_feats, params, edge_index, graph_ids):
    p = params
    src = edge_index[0]
    dst = edge_index[1]
    N, DN = node_feats.shape
    BE = 2000
    BN = 2000

    W1 = p["W_pe1"].T
    W2 = p["W_pe2"].T
    hv_new, c1, ld = _nodeA(
        node_feats, p["W_pn"].T, p["b_pn"].reshape(1, H),
        W1[:DN], p["b_pe1"].reshape(1, H), W2[:H], BN,
    )
    logits, he1 = _edgeB(
        c1[src], edge_feats, ld[dst],
        W1[DN:], W2[H:], p["b_pe2"].reshape(1, 1), BE,
    )
    a = _seg_softmax(logits[:, 0], dst, N)
    q = jax.ops.segment_sum(a[:, None] * he1, dst, num_segments=N)
    sa = jax.ops.segment_sum(a, dst, num_segments=N).reshape(N, 1)
    Wl = p["W_pe"].T
    h, u2, v2, hv_proj = _gruC(
        q, sa, hv_new, p["W_et"].T, p["b_et"].reshape(1, H),
        _split_gru(p["Wih1"], p["Whh1"], p["bih1"], p["bhh1"]),
        Wl[:H], Wl[H:], p["W_pn2"].T, p["b_pn2"].reshape(1, H), BN,
    )

    logits2 = _leaky(u2[dst, 0] + v2[src, 0] + p["b_pe"][0])
    a2 = _seg_softmax(logits2, dst, N)
    ctx2 = jax.ops.segment_sum(hv_proj[src] * a2[:, None], dst, num_segments=N)
    Wz0 = p["Wz0"].T
    Wz1 = p["Wz1"].T
    h2, hz0, hz1 = _gruD(
        ctx2, h, _split_gru(p["Wih2"], p["Whh2"], p["bih2"], p["bhh2"]),
        Wz0[H:], Wz1[H:], BN,
    )

    g_feats = jax.ops.segment_sum(h2, graph_ids, num_segments=G)
    hz = (hz0, hz1)
    Wza = (Wz0[:H], Wz1[:H])
    for t in range(2):
        gz = _gz(g_feats, Wza[t], p["bz%d" % t].reshape(1, 1))
        z = _leaky(gz[graph_ids, 0] + hz[t][:, 0])
        a3 = _seg_softmax(z, graph_ids, G)
        s3h = jax.ops.segment_sum(a3[:, None] * h2, graph_ids, num_segments=G)
        sa3 = jax.ops.segment_sum(a3, graph_ids, num_segments=G).reshape(G, 1)
        g_feats = _gruG(
            s3h, sa3, g_feats, p["Wp%d" % t].T, p["bp%d" % t].reshape(1, H),
            _split_gru(p["Wihr%d" % t], p["Whhr%d" % t], p["bihr%d" % t], p["bhhr%d" % t]),
        )
    return g_feats


# speedup vs baseline: 1.5521x; 1.5521x over previous
import jax
import jax.numpy as jnp
from jax.experimental import pallas as pl

H = 200
G = 256


def _leaky(x):
    return jnp.where(x >= 0, x, 0.01 * x)


def _elu(x):
    return jnp.where(x > 0, x, jnp.exp(jnp.minimum(x, 0.0)) - 1.0)


def _dot(a, b):
    return jnp.dot(a, b, preferred_element_type=jnp.float32)


def _gru_core(x, hh, w):
    wir, wiz, win, whr, whz, whn, bir, biz, bin_, bhr, bhz, bhn = w
    r = jax.nn.sigmoid(_dot(x, wir[...]) + bir[...] + _dot(hh, whr[...]) + bhr[...])
    z = jax.nn.sigmoid(_dot(x, wiz[...]) + biz[...] + _dot(hh, whz[...]) + bhz[...])
    n = jnp.tanh(_dot(x, win[...]) + bin_[...] + r * (_dot(hh, whn[...]) + bhn[...]))
    return jnp.maximum((1.0 - z) * n + z * hh, 0.0)


def _split_gru(Wih, Whh, bih, bhh):
    return (
        Wih[:H].T, Wih[H : 2 * H].T, Wih[2 * H :].T,
        Whh[:H].T, Whh[H : 2 * H].T, Whh[2 * H :].T,
        bih[:H].reshape(1, H), bih[H : 2 * H].reshape(1, H), bih[2 * H :].reshape(1, H),
        bhh[:H].reshape(1, H), bhh[H : 2 * H].reshape(1, H), bhh[2 * H :].reshape(1, H),
    )


def _full(r, c):
    return pl.BlockSpec((r, c), lambda i: (0, 0))


def _row(block, c):
    return pl.BlockSpec((block, c), lambda i: (i, 0))


def _seg_softmax(logits, seg, num_segments):
    m = jax.ops.segment_max(logits, seg, num_segments=num_segments)
    m = jnp.where(jnp.isfinite(m), m, 0.0)
    ex = jnp.exp(logits - m[seg])
    s = jax.ops.segment_sum(ex, seg, num_segments=num_segments)
    return ex / (s[seg] + 1e-12)


def _mm_rows(x, Wt, b, act, block):
    M, K = x.shape
    O = Wt.shape[1]

    def kern(x_ref, w_ref, b_ref, o_ref):
        o_ref[...] = act(_dot(x_ref[...], w_ref[...]) + b_ref[...])

    return pl.pallas_call(
        kern,
        grid=(M // block,),
        in_specs=[_row(block, K), _full(K, O), _full(1, O)],
        out_specs=_row(block, O),
        out_shape=jax.ShapeDtypeStruct((M, O), jnp.float32),
    )(x, Wt, b.reshape(1, O))


def _edge1(nf_src, ef, hv_dst, W1a, W1b, b1, W2a, W2b, b2, block):
    E, DE = ef.shape
    DN = nf_src.shape[1]

    def kern(ns, efr, hd, w1a, w1b, b1r, w2a, w2b, b2r, lo, he_o):
        he1 = _leaky(_dot(ns[...], w1a[...]) + _dot(efr[...], w1b[...]) + b1r[...])
        lo[...] = _leaky(_dot(hd[...], w2a[...]) + _dot(he1, w2b[...]) + b2r[...])
        he_o[...] = he1

    return pl.pallas_call(
        kern,
        grid=(E // block,),
        in_specs=[_row(block, DN), _row(block, DE), _row(block, H),
                  _full(DN, H), _full(DE, H), _full(1, H),
                  _full(H, 1), _full(H, 1), _full(1, 1)],
        out_specs=[_row(block, 1), _row(block, H)],
        out_shape=[
            jax.ShapeDtypeStruct((E, 1), jnp.float32),
            jax.ShapeDtypeStruct((E, H), jnp.float32),
        ],
    )(nf_src, ef, hv_dst, W1a, W1b, b1, W2a, W2b, b2)


def _edge_logits(xa, xb, Wa, Wb, b, block, relu_a=False):
    E = xa.shape[0]

    def kern(ar, br, wa, wb, bb, lo):
        a = ar[...]
        if relu_a:
            a = jnp.maximum(a, 0.0)
        lo[...] = _leaky(_dot(a, wa[...]) + _dot(br[...], wb[...]) + bb[...])

    return pl.pallas_call(
        kern,
        grid=(E // block,),
        in_specs=[_row(block, H), _row(block, H),
                  _full(H, 1), _full(H, 1), _full(1, 1)],
        out_specs=_row(block, 1),
        out_shape=jax.ShapeDtypeStruct((E, 1), jnp.float32),
    )(xa, xb, Wa, Wb, b)


def _gruC(q, sa, hv, Wet, bet, gru_w, Wpn2, bpn2, block):
    N = q.shape[0]

    def kern(qr, sar, hvr, wet, betr, *rest):
        gw = rest[:12]
        wpn2, bpn2r = rest[12:14]
        h_o, hp_o = rest[14:]
        x = _elu(_dot(qr[...], wet[...]) + sar[...] * betr[...])
        h = _gru_core(x, hvr[...], gw)
        h_o[...] = h
        hp_o[...] = _dot(h, wpn2[...]) + bpn2r[...]

    return pl.pallas_call(
        kern,
        grid=(N // block,),
        in_specs=[_row(block, H), _row(block, 1), _row(block, H),
                  _full(H, H), _full(1, H)]
        + [_full(H, H)] * 6 + [_full(1, H)] * 6
        + [_full(H, H), _full(1, H)],
        out_specs=[_row(block, H), _row(block, H)],
        out_shape=[
            jax.ShapeDtypeStruct((N, H), jnp.float32),
            jax.ShapeDtypeStruct((N, H), jnp.float32),
        ],
    )(q, sa, hv, Wet, bet, *gru_w, Wpn2, bpn2)


def _gruD(ctx2, h, gru_w, block):
    N = ctx2.shape[0]

    def kern(cr, hr, *rest):
        gw = rest[:12]
        (h_o,) = rest[12:]
        h_o[...] = _gru_core(_elu(cr[...]), hr[...], gw)

    return pl.pallas_call(
        kern,
        grid=(N // block,),
        in_specs=[_row(block, H), _row(block, H)]
        + [_full(H, H)] * 6 + [_full(1, H)] * 6,
        out_specs=_row(block, H),
        out_shape=jax.ShapeDtypeStruct((N, H), jnp.float32),
    )(ctx2, h, *gru_w)


def _gruG(s3h, sa3, g_feats, Wp, bp, gru_w):
    def kern(sr, sar, gr, wp, bpr, *rest):
        gw = rest[:12]
        (o,) = rest[12:]
        ctx = _elu(_dot(sr[...], wp[...]) + sar[...] * bpr[...])
        o[...] = _gru_core(ctx, gr[...], gw)

    return pl.pallas_call(
        kern,
        grid=(1,),
        in_specs=[_row(G, H), _row(G, 1), _row(G, H), _full(H, H), _full(1, H)]
        + [_full(H, H)] * 6 + [_full(1, H)] * 6,
        out_specs=_row(G, H),
        out_shape=jax.ShapeDtypeStruct((G, H), jnp.float32),
    )(s3h, sa3, g_feats, Wp, bp, *gru_w)


def kernel(node_feats, edge_feats, params, edge_index, graph_ids):
    p = params
    src = edge_index[0]
    dst = edge_index[1]
    N, DN = node_feats.shape
    BE = 2000
    BN = 2000

    hv_new = _mm_rows(node_feats, p["W_pn"].T, p["b_pn"], _leaky, BN)
    W1 = p["W_pe1"].T
    W2 = p["W_pe2"].T
    logits, he1 = _edge1(
        node_feats[src], edge_feats, hv_new[dst],
        W1[:DN], W1[DN:], p["b_pe1"].reshape(1, H),
        W2[:H], W2[H:], p["b_pe2"].reshape(1, 1), BE,
    )
    a = _seg_softmax(logits[:, 0], dst, N)
    q = jax.ops.segment_sum(a[:, None] * he1, dst, num_segments=N)
    sa = jax.ops.segment_sum(a, dst, num_segments=N).reshape(N, 1)
    h, hv_proj = _gruC(
        q, sa, hv_new, p["W_et"].T, p["b_et"].reshape(1, H),
        _split_gru(p["Wih1"], p["Whh1"], p["bih1"], p["bhh1"]),
        p["W_pn2"].T, p["b_pn2"].reshape(1, H), BN,
    )

    Wl = p["W_pe"].T
    logits2 = _edge_logits(h[dst], h[src], Wl[:H], Wl[H:], p["b_pe"].reshape(1, 1), BE)
    a2 = _seg_softmax(logits2[:, 0], dst, N)
    ctx2 = jax.ops.segment_sum(hv_proj[src] * a2[:, None], dst, num_segments=N)
    h2 = _gruD(ctx2, h, _split_gru(p["Wih2"], p["Whh2"], p["bih2"], p["bhh2"]), BN)

    g_feats = jax.ops.segment_sum(h2, graph_ids, num_segments=G)
    for t in range(2):
        Wz = p["Wz%d" % t].T
        z = _edge_logits(
            g_feats[graph_ids], h2, Wz[:H], Wz[H:],
            p["bz%d" % t].reshape(1, 1), BN, relu_a=True,
        )
        a3 = _seg_softmax(z[:, 0], graph_ids, G)
        s3h = jax.ops.segment_sum(a3[:, None] * h2, graph_ids, num_segments=G)
        sa3 = jax.ops.segment_sum(a3, graph_ids, num_segments=G).reshape(G, 1)
        g_feats = _gruG(
            s3h, sa3, g_feats, p["Wp%d" % t].T, p["bp%d" % t].reshape(1, H),
            _split_gru(p["Wihr%d" % t], p["Whhr%d" % t], p["bihr%d" % t], p["bhhr%d" % t]),
        )
    return g_feats


# speedup vs baseline: 2.0404x; 1.3146x over previous
import jax
import jax.numpy as jnp
from jax.experimental import pallas as pl

H = 200
G = 256


def _leaky(x):
    return jnp.where(x >= 0, x, 0.01 * x)


def _elu(x):
    return jnp.where(x > 0, x, jnp.exp(jnp.minimum(x, 0.0)) - 1.0)


def _dot(a, b):
    return jnp.dot(a, b, preferred_element_type=jnp.float32)


def _gru_core(x, hh, w):
    wir, wiz, win, whr, whz, whn, bir, biz, bin_, bhr, bhz, bhn = w
    r = jax.nn.sigmoid(_dot(x, wir[...]) + bir[...] + _dot(hh, whr[...]) + bhr[...])
    z = jax.nn.sigmoid(_dot(x, wiz[...]) + biz[...] + _dot(hh, whz[...]) + bhz[...])
    n = jnp.tanh(_dot(x, win[...]) + bin_[...] + r * (_dot(hh, whn[...]) + bhn[...]))
    return jnp.maximum((1.0 - z) * n + z * hh, 0.0)


def _split_gru(Wih, Whh, bih, bhh):
    return (
        Wih[:H].T, Wih[H : 2 * H].T, Wih[2 * H :].T,
        Whh[:H].T, Whh[H : 2 * H].T, Whh[2 * H :].T,
        bih[:H].reshape(1, H), bih[H : 2 * H].reshape(1, H), bih[2 * H :].reshape(1, H),
        bhh[:H].reshape(1, H), bhh[H : 2 * H].reshape(1, H), bhh[2 * H :].reshape(1, H),
    )


def _full(r, c):
    return pl.BlockSpec((r, c), lambda i: (0, 0))


def _row(block, c):
    return pl.BlockSpec((block, c), lambda i: (i, 0))


def _seg_softmax(logits, seg, num_segments):
    m = jax.ops.segment_max(logits, seg, num_segments=num_segments)
    m = jnp.where(jnp.isfinite(m), m, 0.0)
    ex = jnp.exp(logits - m[seg])
    s = jax.ops.segment_sum(ex, seg, num_segments=num_segments)
    return ex / (s[seg] + 1e-12)


def _mm_rows(x, Wt, b, act, block):
    M, K = x.shape
    O = Wt.shape[1]

    def kern(x_ref, w_ref, b_ref, o_ref):
        o_ref[...] = act(_dot(x_ref[...], w_ref[...]) + b_ref[...])

    return pl.pallas_call(
        kern,
        grid=(M // block,),
        in_specs=[_row(block, K), _full(K, O), _full(1, O)],
        out_specs=_row(block, O),
        out_shape=jax.ShapeDtypeStruct((M, O), jnp.float32),
    )(x, Wt, b.reshape(1, O))


def _edge1(nf_src, ef, hv_dst, W1a, W1b, b1, W2a, W2b, b2, block):
    E, DE = ef.shape
    DN = nf_src.shape[1]

    def kern(ns, efr, hd, w1a, w1b, b1r, w2a, w2b, b2r, lo, he_o):
        he1 = _leaky(_dot(ns[...], w1a[...]) + _dot(efr[...], w1b[...]) + b1r[...])
        lo[...] = _leaky(_dot(hd[...], w2a[...]) + _dot(he1, w2b[...]) + b2r[...])
        he_o[...] = he1

    return pl.pallas_call(
        kern,
        grid=(E // block,),
        in_specs=[_row(block, DN), _row(block, DE), _row(block, H),
                  _full(DN, H), _full(DE, H), _full(1, H),
                  _full(H, 1), _full(H, 1), _full(1, 1)],
        out_specs=[_row(block, 1), _row(block, H)],
        out_shape=[
            jax.ShapeDtypeStruct((E, 1), jnp.float32),
            jax.ShapeDtypeStruct((E, H), jnp.float32),
        ],
    )(nf_src, ef, hv_dst, W1a, W1b, b1, W2a, W2b, b2)


def _edge_logits(xa, xb, Wa, Wb, b, block, relu_a=False):
    E = xa.shape[0]

    def kern(ar, br, wa, wb, bb, lo):
        a = ar[...]
        if relu_a:
            a = jnp.maximum(a, 0.0)
        lo[...] = _leaky(_dot(a, wa[...]) + _dot(br[...], wb[...]) + bb[...])

    return pl.pallas_call(
        kern,
        grid=(E // block,),
        in_specs=[_row(block, H), _row(block, H),
                  _full(H, 1), _full(H, 1), _full(1, 1)],
        out_specs=_row(block, 1),
        out_shape=jax.ShapeDtypeStruct((E, 1), jnp.float32),
    )(xa, xb, Wa, Wb, b)


def _gruC(q, sa, hv, Wet, bet, gru_w, Wpn2, bpn2, block):
    N = q.shape[0]

    def kern(qr, sar, hvr, wet, betr, *rest):
        gw = rest[:12]
        wpn2, bpn2r = rest[12:14]
        h_o, hp_o = rest[14:]
        s = sar[...]
        x = _elu((_dot(qr[...], wet[...]) + s * betr[...]) / (s + 1e-12))
        h = _gru_core(x, hvr[...], gw)
        h_o[...] = h
        hp_o[...] = _dot(h, wpn2[...]) + bpn2r[...]

    return pl.pallas_call(
        kern,
        grid=(N // block,),
        in_specs=[_row(block, H), _row(block, 1), _row(block, H),
                  _full(H, H), _full(1, H)]
        + [_full(H, H)] * 6 + [_full(1, H)] * 6
        + [_full(H, H), _full(1, H)],
        out_specs=[_row(block, H), _row(block, H)],
        out_shape=[
            jax.ShapeDtypeStruct((N, H), jnp.float32),
            jax.ShapeDtypeStruct((N, H), jnp.float32),
        ],
    )(q, sa, hv, Wet, bet, *gru_w, Wpn2, bpn2)


def _gruD(c2n, s2, h, gru_w, block):
    N = c2n.shape[0]

    def kern(cr, sr, hr, *rest):
        gw = rest[:12]
        (h_o,) = rest[12:]
        h_o[...] = _gru_core(_elu(cr[...] / (sr[...] + 1e-12)), hr[...], gw)

    return pl.pallas_call(
        kern,
        grid=(N // block,),
        in_specs=[_row(block, H), _row(block, 1), _row(block, H)]
        + [_full(H, H)] * 6 + [_full(1, H)] * 6,
        out_specs=_row(block, H),
        out_shape=jax.ShapeDtypeStruct((N, H), jnp.float32),
    )(c2n, s2, h, *gru_w)


def _gruG(s3h, sa3, g_feats, Wp, bp, gru_w):
    def kern(sr, sar, gr, wp, bpr, *rest):
        gw = rest[:12]
        (o,) = rest[12:]
        s = sar[...]
        ctx = _elu((_dot(sr[...], wp[...]) + s * bpr[...]) / (s + 1e-12))
        o[...] = _gru_core(ctx, gr[...], gw)

    return pl.pallas_call(
        kern,
        grid=(1,),
        in_specs=[_row(G, H), _row(G, 1), _row(G, H), _full(H, H), _full(1, H)]
        + [_full(H, H)] * 6 + [_full(1, H)] * 6,
        out_specs=_row(G, H),
        out_shape=jax.ShapeDtypeStruct((G, H), jnp.float32),
    )(s3h, sa3, g_feats, Wp, bp, *gru_w)


def kernel(node_feats, edge_feats, params, edge_index, graph_ids):
    p = params
    src = edge_index[0]
    dst = edge_index[1]
    N, DN = node_feats.shape
    BE = 2000
    BN = 2000

    hv_new = _mm_rows(node_feats, p["W_pn"].T, p["b_pn"], _leaky, BN)
    W1 = p["W_pe1"].T
    W2 = p["W_pe2"].T
    logits, he1 = _edge1(
        node_feats[src], edge_feats, hv_new[dst],
        W1[:DN], W1[DN:], p["b_pe1"].reshape(1, H),
        W2[:H], W2[H:], p["b_pe2"].reshape(1, 1), BE,
    )
    m = jax.ops.segment_max(logits[:, 0], dst, num_segments=N)
    m = jnp.where(jnp.isfinite(m), m, 0.0)
    ex = jnp.exp(logits[:, 0] - m[dst])
    qn = jax.ops.segment_sum(ex[:, None] * he1, dst, num_segments=N)
    s = jax.ops.segment_sum(ex, dst, num_segments=N).reshape(N, 1)
    h, hv_proj = _gruC(
        qn, s, hv_new, p["W_et"].T, p["b_et"].reshape(1, H),
        _split_gru(p["Wih1"], p["Whh1"], p["bih1"], p["bhh1"]),
        p["W_pn2"].T, p["b_pn2"].reshape(1, H), BN,
    )

    Wl = p["W_pe"].T
    logits2 = _edge_logits(h[dst], h[src], Wl[:H], Wl[H:], p["b_pe"].reshape(1, 1), BE)
    m2 = jax.ops.segment_max(logits2[:, 0], dst, num_segments=N)
    m2 = jnp.where(jnp.isfinite(m2), m2, 0.0)
    ex2 = jnp.exp(logits2[:, 0] - m2[dst])
    c2n = jax.ops.segment_sum(ex2[:, None] * hv_proj[src], dst, num_segments=N)
    s2 = jax.ops.segment_sum(ex2, dst, num_segments=N).reshape(N, 1)
    h2 = _gruD(c2n, s2, h, _split_gru(p["Wih2"], p["Whh2"], p["bih2"], p["bhh2"]), BN)

    g_feats = jax.ops.segment_sum(h2, graph_ids, num_segments=G)
    for t in range(2):
        Wz = p["Wz%d" % t].T
        z = _edge_logits(
            g_feats[graph_ids], h2, Wz[:H], Wz[H:],
            p["bz%d" % t].reshape(1, 1), BN, relu_a=True,
        )
        m3 = jax.ops.segment_max(z[:, 0], graph_ids, num_segments=G)
        m3 = jnp.where(jnp.isfinite(m3), m3, 0.0)
        ex3 = jnp.exp(z[:, 0] - m3[graph_ids])
        s3n = jax.ops.segment_sum(ex3[:, None] * h2, graph_ids, num_segments=G)
        s3 = jax.ops.segment_sum(ex3, graph_ids, num_segments=G).reshape(G, 1)
        g_feats = _gruG(
            s3n, s3, g_feats, p["Wp%d" % t].T, p["bp%d" % t].reshape(1, H),
            _split_gru(p["Wihr%d" % t], p["Whhr%d" % t], p["bihr%d" % t], p["bhhr%d" % t]),
        )
    return g_feats


# speedup vs baseline: 2.1458x; 1.0516x over previous
import jax
import jax.numpy as jnp
from jax.experimental import pallas as pl

H = 200
G = 256


def _leaky(x):
    return jnp.where(x >= 0, x, 0.01 * x)


def _elu(x):
    return jnp.where(x > 0, x, jnp.exp(jnp.minimum(x, 0.0)) - 1.0)


def _dot(a, b):
    return jnp.dot(a, b, preferred_element_type=jnp.float32)


def _gru_core(x, hh, w):
    wir, wiz, win, whr, whz, whn, bir, biz, bin_, bhr, bhz, bhn = w
    r = jax.nn.sigmoid(_dot(x, wir[...]) + bir[...] + _dot(hh, whr[...]) + bhr[...])
    z = jax.nn.sigmoid(_dot(x, wiz[...]) + biz[...] + _dot(hh, whz[...]) + bhz[...])
    n = jnp.tanh(_dot(x, win[...]) + bin_[...] + r * (_dot(hh, whn[...]) + bhn[...]))
    return jnp.maximum((1.0 - z) * n + z * hh, 0.0)


def _split_gru(Wih, Whh, bih, bhh):
    return (
        Wih[:H].T, Wih[H : 2 * H].T, Wih[2 * H :].T,
        Whh[:H].T, Whh[H : 2 * H].T, Whh[2 * H :].T,
        bih[:H].reshape(1, H), bih[H : 2 * H].reshape(1, H), bih[2 * H :].reshape(1, H),
        bhh[:H].reshape(1, H), bhh[H : 2 * H].reshape(1, H), bhh[2 * H :].reshape(1, H),
    )


def _full(r, c):
    return pl.BlockSpec((r, c), lambda i: (0, 0))


def _row(block, c):
    return pl.BlockSpec((block, c), lambda i: (i, 0))


def _seg_softmax(logits, seg, num_segments):
    m = jax.ops.segment_max(logits, seg, num_segments=num_segments)
    m = jnp.where(jnp.isfinite(m), m, 0.0)
    ex = jnp.exp(logits - m[seg])
    s = jax.ops.segment_sum(ex, seg, num_segments=num_segments)
    return ex / (s[seg] + 1e-12)


def _mm_rows(x, Wt, b, act, block):
    M, K = x.shape
    O = Wt.shape[1]

    def kern(x_ref, w_ref, b_ref, o_ref):
        o_ref[...] = act(_dot(x_ref[...], w_ref[...]) + b_ref[...])

    return pl.pallas_call(
        kern,
        grid=(M // block,),
        in_specs=[_row(block, K), _full(K, O), _full(1, O)],
        out_specs=_row(block, O),
        out_shape=jax.ShapeDtypeStruct((M, O), jnp.float32),
    )(x, Wt, b.reshape(1, O))


def _edge1(nf_src, ef, hv_dst, W1a, W1b, b1, W2a, W2b, b2, block):
    E, DE = ef.shape
    DN = nf_src.shape[1]

    def kern(ns, efr, hd, w1a, w1b, b1r, w2a, w2b, b2r, lo, he_o):
        he1 = _leaky(_dot(ns[...], w1a[...]) + _dot(efr[...], w1b[...]) + b1r[...])
        lo[...] = _leaky(_dot(hd[...], w2a[...]) + _dot(he1, w2b[...]) + b2r[...])
        he_o[...] = he1

    return pl.pallas_call(
        kern,
        grid=(E // block,),
        in_specs=[_row(block, DN), _row(block, DE), _row(block, H),
                  _full(DN, H), _full(DE, H), _full(1, H),
                  _full(H, 1), _full(H, 1), _full(1, 1)],
        out_specs=[_row(block, 1), _row(block, H)],
        out_shape=[
            jax.ShapeDtypeStruct((E, 1), jnp.float32),
            jax.ShapeDtypeStruct((E, H), jnp.float32),
        ],
    )(nf_src, ef, hv_dst, W1a, W1b, b1, W2a, W2b, b2)


def _edge_logits_bias(xa, sb, Wa, b, block):
    E = xa.shape[0]

    def kern(ar, sbr, wa, bb, lo):
        lo[...] = _leaky(_dot(ar[...], wa[...]) + sbr[...] + bb[...])

    return pl.pallas_call(
        kern,
        grid=(E // block,),
        in_specs=[_row(block, H), _row(block, 1), _full(H, 1), _full(1, 1)],
        out_specs=_row(block, 1),
        out_shape=jax.ShapeDtypeStruct((E, 1), jnp.float32),
    )(xa, sb, Wa, b)


def _edge_logits(xa, xb, Wa, Wb, b, block, relu_a=False):
    E = xa.shape[0]

    def kern(ar, br, wa, wb, bb, lo):
        a = ar[...]
        if relu_a:
            a = jnp.maximum(a, 0.0)
        lo[...] = _leaky(_dot(a, wa[...]) + _dot(br[...], wb[...]) + bb[...])

    return pl.pallas_call(
        kern,
        grid=(E // block,),
        in_specs=[_row(block, H), _row(block, H),
                  _full(H, 1), _full(H, 1), _full(1, 1)],
        out_specs=_row(block, 1),
        out_shape=jax.ShapeDtypeStruct((E, 1), jnp.float32),
    )(xa, xb, Wa, Wb, b)


def _gruC(q, sa, hv, Wet, bet, gru_w, Wpn2, bpn2, Wlb, block):
    N = q.shape[0]

    def kern(qr, sar, hvr, wet, betr, *rest):
        gw = rest[:12]
        wpn2, bpn2r, wlb = rest[12:15]
        h_o, hp_o, v_o = rest[15:]
        s = sar[...]
        x = _elu((_dot(qr[...], wet[...]) + s * betr[...]) / (s + 1e-12))
        h = _gru_core(x, hvr[...], gw)
        h_o[...] = h
        hp_o[...] = _dot(h, wpn2[...]) + bpn2r[...]
        v_o[...] = _dot(h, wlb[...])

    return pl.pallas_call(
        kern,
        grid=(N // block,),
        in_specs=[_row(block, H), _row(block, 1), _row(block, H),
                  _full(H, H), _full(1, H)]
        + [_full(H, H)] * 6 + [_full(1, H)] * 6
        + [_full(H, H), _full(1, H), _full(H, 1)],
        out_specs=[_row(block, H), _row(block, H), _row(block, 1)],
        out_shape=[
            jax.ShapeDtypeStruct((N, H), jnp.float32),
            jax.ShapeDtypeStruct((N, H), jnp.float32),
            jax.ShapeDtypeStruct((N, 1), jnp.float32),
        ],
    )(q, sa, hv, Wet, bet, *gru_w, Wpn2, bpn2, Wlb)


def _gruD(c2n, s2, h, gru_w, block):
    N = c2n.shape[0]

    def kern(cr, sr, hr, *rest):
        gw = rest[:12]
        (h_o,) = rest[12:]
        h_o[...] = _gru_core(_elu(cr[...] / (sr[...] + 1e-12)), hr[...], gw)

    return pl.pallas_call(
        kern,
        grid=(N // block,),
        in_specs=[_row(block, H), _row(block, 1), _row(block, H)]
        + [_full(H, H)] * 6 + [_full(1, H)] * 6,
        out_specs=_row(block, H),
        out_shape=jax.ShapeDtypeStruct((N, H), jnp.float32),
    )(c2n, s2, h, *gru_w)


def _gruG(s3h, sa3, g_feats, Wp, bp, gru_w):
    def kern(sr, sar, gr, wp, bpr, *rest):
        gw = rest[:12]
        (o,) = rest[12:]
        s = sar[...]
        ctx = _elu((_dot(sr[...], wp[...]) + s * bpr[...]) / (s + 1e-12))
        o[...] = _gru_core(ctx, gr[...], gw)

    return pl.pallas_call(
        kern,
        grid=(1,),
        in_specs=[_row(G, H), _row(G, 1), _row(G, H), _full(H, H), _full(1, H)]
        + [_full(H, H)] * 6 + [_full(1, H)] * 6,
        out_specs=_row(G, H),
        out_shape=jax.ShapeDtypeStruct((G, H), jnp.float32),
    )(s3h, sa3, g_feats, Wp, bp, *gru_w)


def kernel(node_feats, edge_feats, params, edge_index, graph_ids):
    p = params
    src = edge_index[0]
    dst = edge_index[1]
    N, DN = node_feats.shape
    BE = 2000
    BN = 2000

    hv_new = _mm_rows(node_feats, p["W_pn"].T, p["b_pn"], _leaky, BN)
    W1 = p["W_pe1"].T
    W2 = p["W_pe2"].T
    logits, he1 = _edge1(
        node_feats[src], edge_feats, hv_new[dst],
        W1[:DN], W1[DN:], p["b_pe1"].reshape(1, H),
        W2[:H], W2[H:], p["b_pe2"].reshape(1, 1), BE,
    )
    m = jax.ops.segment_max(logits[:, 0], dst, num_segments=N)
    m = jnp.where(jnp.isfinite(m), m, 0.0)
    ex = jnp.exp(logits[:, 0] - m[dst])
    qn = jax.ops.segment_sum(ex[:, None] * he1, dst, num_segments=N)
    s = jax.ops.segment_sum(ex, dst, num_segments=N).reshape(N, 1)
    Wl = p["W_pe"].T
    h, hv_proj, v2 = _gruC(
        qn, s, hv_new, p["W_et"].T, p["b_et"].reshape(1, H),
        _split_gru(p["Wih1"], p["Whh1"], p["bih1"], p["bhh1"]),
        p["W_pn2"].T, p["b_pn2"].reshape(1, H), Wl[H:], BN,
    )

    ge = jnp.concatenate([hv_proj, v2], axis=1)[src]
    logits2 = _edge_logits_bias(h[dst], ge[:, H:], Wl[:H], p["b_pe"].reshape(1, 1), BE)
    m2 = jax.ops.segment_max(logits2[:, 0], dst, num_segments=N)
    m2 = jnp.where(jnp.isfinite(m2), m2, 0.0)
    ex2 = jnp.exp(logits2[:, 0] - m2[dst])
    c2n = jax.ops.segment_sum(ex2[:, None] * ge[:, :H], dst, num_segments=N)
    s2 = jax.ops.segment_sum(ex2, dst, num_segments=N).reshape(N, 1)
    h2 = _gruD(c2n, s2, h, _split_gru(p["Wih2"], p["Whh2"], p["bih2"], p["bhh2"]), BN)

    g_feats = jax.ops.segment_sum(h2, graph_ids, num_segments=G)
    for t in range(2):
        Wz = p["Wz%d" % t].T
        z = _edge_logits(
            g_feats[graph_ids], h2, Wz[:H], Wz[H:],
            p["bz%d" % t].reshape(1, 1), BN, relu_a=True,
        )
        m3 = jax.ops.segment_max(z[:, 0], graph_ids, num_segments=G)
        m3 = jnp.where(jnp.isfinite(m3), m3, 0.0)
        ex3 = jnp.exp(z[:, 0] - m3[graph_ids])
        s3n = jax.ops.segment_sum(ex3[:, None] * h2, graph_ids, num_segments=G)
        s3 = jax.ops.segment_sum(ex3, graph_ids, num_segments=G).reshape(G, 1)
        g_feats = _gruG(
            s3n, s3, g_feats, p["Wp%d" % t].T, p["bp%d" % t].reshape(1, H),
            _split_gru(p["Wihr%d" % t], p["Whhr%d" % t], p["bihr%d" % t], p["bhhr%d" % t]),
        )
    return g_feats
